# baseline (speedup 1.0000x reference)
import functools

import numpy as np
import jax
import jax.numpy as jnp
from jax import lax
from jax.experimental import pallas as pl
from jax.experimental.pallas import tpu as pltpu
from jax.experimental.pallas import tpu_sc as plsc

_B = 16384
_K = 100
_SPAN = 999999

_ROT = ((13, 15, 26, 6), (17, 29, 16, 24))


_K2_0, _K2_1 = 64467757, 2916123636


def _key_schedule():
    m = (1 << 32) - 1
    ks = (_K2_0, _K2_1, _K2_0 ^ _K2_1 ^ 0x1BD11BDA)
    return tuple(
        (ks[(r + 1) % 3], (ks[(r + 2) % 3] + r + 1) & m) for r in range(5)
    )


_KS = _key_schedule()


def _neg_sample_block(o_ref, *, rows_per_block, cols):
    base = pl.program_id(0) * (rows_per_block * cols)
    shape = (rows_per_block, cols)
    i = (jax.lax.broadcasted_iota(jnp.int32, shape, 0) * cols
         + jax.lax.broadcasted_iota(jnp.int32, shape, 1)
         + base).astype(jnp.uint32)

    x0 = jnp.full(shape, _K2_0, jnp.uint32)
    x1 = i + jnp.uint32(_K2_1)
    for r in range(5):
        for d in _ROT[r % 2]:
            x0 = x0 + x1
            x1 = (x1 << d) | (x1 >> (32 - d))
            x1 = x0 ^ x1
        x0 = x0 + jnp.uint32(_KS[r][0])
        x1 = x1 + jnp.uint32(_KS[r][1])
    bits = x0 ^ x1

    qf = (bits >> 1).astype(jnp.int32).astype(jnp.float32) * (2.0 / float(_SPAN))
    q = qf.astype(jnp.int32).astype(jnp.uint32)
    r = bits - q * jnp.uint32(_SPAN)
    r = jnp.where(r >= jnp.uint32(0x80000000), r + jnp.uint32(_SPAN), r)
    r = jnp.where(r >= jnp.uint32(_SPAN), r - jnp.uint32(_SPAN), r)
    o_ref[...] = (r + jnp.uint32(1)).astype(jnp.int32)


def _tf16(i):
    x0 = jnp.full((16,), _K2_0, jnp.uint32)
    x1 = i + jnp.uint32(_K2_1)
    for r in range(5):
        for d in _ROT[r % 2]:
            x0 = x0 + x1
            x1 = (x1 << d) | (x1 >> (32 - d))
            x1 = x0 ^ x1
        x0 = x0 + jnp.uint32(_KS[r][0])
        x1 = x1 + jnp.uint32(_KS[r][1])
    bits = x0 ^ x1
    qf = (bits >> 1).astype(jnp.int32).astype(jnp.float32) * (2.0 / float(_SPAN))
    q = qf.astype(jnp.int32).astype(jnp.uint32)
    r = bits - q * jnp.uint32(_SPAN)
    r = jnp.where(r >= jnp.uint32(0x80000000), r + jnp.uint32(_SPAN), r)
    r = jnp.where(r >= jnp.uint32(_SPAN), r - jnp.uint32(_SPAN), r)
    return (r + jnp.uint32(1)).astype(jnp.int32)


_SC_TILES = 32
_SC_COLS = 400


def _sc_rows_kernel(n_rows):
    rows_per_tile = n_rows // _SC_TILES
    mesh = plsc.VectorSubcoreMesh(core_axis_name="c", subcore_axis_name="s")

    @functools.partial(
        pl.kernel,
        mesh=mesh,
        out_type=jax.ShapeDtypeStruct((n_rows, _SC_COLS), jnp.int32),
        scratch_types=[
            pltpu.VMEM((rows_per_tile, _SC_COLS), jnp.int32),
            pltpu.SemaphoreType.DMA,
        ],
    )
    def sc_kernel(out_hbm, buf, sem):
        wid = lax.axis_index("s") * 2 + lax.axis_index("c")
        row0 = wid * rows_per_tile
        lanes = lax.iota(jnp.int32, 16).astype(jnp.uint32)

        def row_body(rr, _):
            base = (row0 + rr) * _SC_COLS
            for v in range(_SC_COLS // 16):
                i = lanes + jnp.uint32(base + v * 16)
                buf[rr, pl.ds(v * 16, 16)] = _tf16(i)
            return 0

        lax.fori_loop(0, rows_per_tile, row_body, 0)
        pltpu.async_copy(buf, out_hbm.at[pl.ds(row0, rows_per_tile)], sem).wait()

    return sc_kernel


def kernel(k, pos_targets):
    del k, pos_targets
    sc_out = _sc_rows_kernel(_B * _K // _SC_COLS)()
    return sc_out.reshape(_B, _K)


# speedup vs baseline: 2.0259x; 2.0259x over previous
import functools

import numpy as np
import jax
import jax.numpy as jnp
from jax import lax
from jax.experimental import pallas as pl
from jax.experimental.pallas import tpu as pltpu
from jax.experimental.pallas import tpu_sc as plsc

_B = 16384
_K = 100
_SPAN = 999999

_ROT = ((13, 15, 26, 6), (17, 29, 16, 24))


_K2_0, _K2_1 = 64467757, 2916123636


def _key_schedule():
    m = (1 << 32) - 1
    ks = (_K2_0, _K2_1, _K2_0 ^ _K2_1 ^ 0x1BD11BDA)
    return tuple(
        (ks[(r + 1) % 3], (ks[(r + 2) % 3] + r + 1) & m) for r in range(5)
    )


_KS = _key_schedule()


def _neg_sample_block(base_ref, o_ref, *, rows_per_block, cols):
    base = (base_ref[0] + pl.program_id(0) * rows_per_block) * cols
    shape = (rows_per_block, cols)
    i = (jax.lax.broadcasted_iota(jnp.int32, shape, 0) * cols
         + jax.lax.broadcasted_iota(jnp.int32, shape, 1)
         + base).astype(jnp.uint32)

    x0 = jnp.full(shape, _K2_0, jnp.uint32)
    x1 = i + jnp.uint32(_K2_1)
    for r in range(5):
        for d in _ROT[r % 2]:
            x0 = x0 + x1
            x1 = (x1 << d) | (x1 >> (32 - d))
            x1 = x0 ^ x1
        x0 = x0 + jnp.uint32(_KS[r][0])
        x1 = x1 + jnp.uint32(_KS[r][1])
    bits = x0 ^ x1

    qf = (bits >> 1).astype(jnp.int32).astype(jnp.float32) * (2.0 / float(_SPAN))
    q = qf.astype(jnp.int32).astype(jnp.uint32)
    r = bits - q * jnp.uint32(_SPAN)
    r = jnp.where(r >= jnp.uint32(0x80000000), r + jnp.uint32(_SPAN), r)
    r = jnp.where(r >= jnp.uint32(_SPAN), r - jnp.uint32(_SPAN), r)
    o_ref[...] = (r + jnp.uint32(1)).astype(jnp.int32)


def _rows_call(n_rows, rows_per_block, base_arr):
    grid = (n_rows // rows_per_block,)
    return pl.pallas_call(
        functools.partial(_neg_sample_block, rows_per_block=rows_per_block,
                          cols=_K),
        grid=grid,
        in_specs=[pl.BlockSpec(memory_space=pltpu.SMEM)],
        out_shape=jax.ShapeDtypeStruct((n_rows, _K), jnp.int32),
        out_specs=pl.BlockSpec((rows_per_block, _K), lambda b: (b, 0)),
    )(base_arr)


def kernel(k, pos_targets):
    del k, pos_targets
    devs = jax.devices()
    n_dev = 1
    while n_dev * 2 <= len(devs) and _B % (n_dev * 2) == 0:
        n_dev *= 2
    if n_dev == 1:
        return _rows_call(_B, 2048, jnp.zeros((1,), jnp.int32))

    from jax.sharding import Mesh, PartitionSpec as P
    from jax.experimental.shard_map import shard_map

    rows_local = _B // n_dev
    mesh = Mesh(np.array(devs[:n_dev]), ("d",))

    def shard_fn():
        base = (lax.axis_index("d") * rows_local).astype(jnp.int32)
        return _rows_call(rows_local, min(2048, rows_local), base[None])

    f = shard_map(shard_fn, mesh=mesh, in_specs=(), out_specs=P("d", None),
                  check_rep=False)
    return f()


# speedup vs baseline: 2.6328x; 1.2995x over previous
import functools

import numpy as np
import jax
import jax.numpy as jnp
from jax import lax
from jax.experimental import pallas as pl
from jax.experimental.pallas import tpu as pltpu
from jax.experimental.pallas import tpu_sc as plsc

_B = 16384
_K = 100
_SPAN = 999999

_ROT = ((13, 15, 26, 6), (17, 29, 16, 24))


_K2_0, _K2_1 = 64467757, 2916123636


def _key_schedule():
    m = (1 << 32) - 1
    ks = (_K2_0, _K2_1, _K2_0 ^ _K2_1 ^ 0x1BD11BDA)
    return tuple(
        (ks[(r + 1) % 3], (ks[(r + 2) % 3] + r + 1) & m) for r in range(5)
    )


_KS = _key_schedule()


def _neg_sample_block(base_ref, o_ref, *, rows_per_block, cols):
    base = (base_ref[0] + pl.program_id(0) * rows_per_block) * cols
    shape = (rows_per_block, cols)
    i = (jax.lax.broadcasted_iota(jnp.int32, shape, 0) * cols
         + jax.lax.broadcasted_iota(jnp.int32, shape, 1)
         + base).astype(jnp.uint32)

    x0 = jnp.full(shape, _K2_0, jnp.uint32)
    x1 = i + jnp.uint32(_K2_1)
    for r in range(5):
        for d in _ROT[r % 2]:
            x0 = x0 + x1
            x1 = (x1 << d) | (x1 >> (32 - d))
            x1 = x0 ^ x1
        x0 = x0 + jnp.uint32(_KS[r][0])
        x1 = x1 + jnp.uint32(_KS[r][1])
    bits = x0 ^ x1

    qf = (bits >> 1).astype(jnp.int32).astype(jnp.float32) * (2.0 / float(_SPAN))
    q = qf.astype(jnp.int32).astype(jnp.uint32)
    r = bits - q * jnp.uint32(_SPAN)
    r = jnp.where(r >= jnp.uint32(0x80000000), r + jnp.uint32(_SPAN), r)
    r = jnp.where(r >= jnp.uint32(_SPAN), r - jnp.uint32(_SPAN), r)
    o_ref[...] = (r + jnp.uint32(1)).astype(jnp.int32)


def _rows_call(n_rows, rows_per_block, base_arr):
    grid = (n_rows // rows_per_block,)
    return pl.pallas_call(
        functools.partial(_neg_sample_block, rows_per_block=rows_per_block,
                          cols=_K),
        grid=grid,
        in_specs=[pl.BlockSpec(memory_space=pltpu.SMEM)],
        out_shape=jax.ShapeDtypeStruct((n_rows, _K), jnp.int32),
        out_specs=pl.BlockSpec((rows_per_block, _K), lambda b: (b, 0)),
    )(base_arr)


def kernel(k, pos_targets):
    del k, pos_targets
    return _rows_call(_B, 1024, jnp.zeros((1,), jnp.int32))


# speedup vs baseline: 2.7182x; 1.0325x over previous
import functools

import numpy as np
import jax
import jax.numpy as jnp
from jax import lax
from jax.experimental import pallas as pl
from jax.experimental.pallas import tpu as pltpu
from jax.experimental.pallas import tpu_sc as plsc

_B = 16384
_K = 100
_SPAN = 999999

_ROT = ((13, 15, 26, 6), (17, 29, 16, 24))


_K2_0, _K2_1 = 64467757, 2916123636


def _key_schedule():
    m = (1 << 32) - 1
    ks = (_K2_0, _K2_1, _K2_0 ^ _K2_1 ^ 0x1BD11BDA)
    return tuple(
        (ks[(r + 1) % 3], (ks[(r + 2) % 3] + r + 1) & m) for r in range(5)
    )


_KS = _key_schedule()


def _neg_sample_block(base_ref, o_ref, *, rows_per_block, cols):
    base = (base_ref[0] + pl.program_id(0) * rows_per_block) * cols
    shape = (rows_per_block, cols)
    scal = base.astype(jnp.uint32) + jnp.uint32(_K2_1)
    x1 = (jax.lax.broadcasted_iota(jnp.uint32, shape, 0) * jnp.uint32(cols)
          + jax.lax.broadcasted_iota(jnp.uint32, shape, 1)
          + scal)

    x0 = jnp.full(shape, _K2_0, jnp.uint32)
    for r in range(5):
        for d in _ROT[r % 2]:
            x0 = x0 + x1
            x1 = (x1 << d) | (x1 >> (32 - d))
            x1 = x0 ^ x1
        x0 = x0 + jnp.uint32(_KS[r][0])
        x1 = x1 + jnp.uint32(_KS[r][1])
    bits = x0 ^ x1
    r = bits % jnp.uint32(_SPAN)
    o_ref[...] = (r + jnp.uint32(1)).astype(jnp.int32)


def _rows_call(n_rows, rows_per_block, base_arr):
    grid = (n_rows // rows_per_block,)
    return pl.pallas_call(
        functools.partial(_neg_sample_block, rows_per_block=rows_per_block,
                          cols=_K),
        grid=grid,
        in_specs=[pl.BlockSpec(memory_space=pltpu.SMEM)],
        out_shape=jax.ShapeDtypeStruct((n_rows, _K), jnp.int32),
        out_specs=pl.BlockSpec((rows_per_block, _K), lambda b: (b, 0)),
    )(base_arr)


def kernel(k, pos_targets):
    del k, pos_targets
    return _rows_call(_B, 1024, jnp.zeros((1,), jnp.int32))
